# baseline (speedup 1.0000x reference)
import functools

import numpy as np
import jax
import jax.numpy as jnp
from jax import lax
from jax.experimental import pallas as pl
from jax.experimental.pallas import tpu as pltpu
from jax.experimental.pallas import tpu_sc as plsc

_LANES = 128

_K0 = np.uint32(908003072)
_K1 = np.uint32(3252900185)
_K2 = np.uint32(_K0 ^ _K1 ^ np.uint32(0x1BD11BDA))
_KS = (_K0, _K1, _K2)
_ROT = ((13, 15, 26, 6), (17, 29, 16, 24))

_SPAN = np.float32(np.float32(1.0) - np.float32(1e-10))
_MINV = np.float32(1e-10)

_LOG2_COEFFS = (
    1.4426950215754873, -0.7213459745244996, 0.4808460790740989,
    -0.3599757274729275, 0.2836810758200936, -0.2202947345679977,
    0.1521123404570871, -0.0813299297601238, 0.0281939127697494,
    -0.0045820805842636,
)
_LN2 = np.float32(0.6931471805599453)


def _threefry_bits(x1v):
    x0 = jnp.full_like(x1v, _K0)
    x1 = x1v + _K1
    for i in range(5):
        for r in _ROT[i % 2]:
            x0 = x0 + x1
            x1 = (x1 << r) | (x1 >> (32 - r))
            x1 = x0 ^ x1
        x0 = x0 + _KS[(i + 1) % 3]
        x1 = x1 + np.uint32(_KS[(i + 2) % 3] + np.uint32(i + 1))
    return x0 ^ x1


def _unit(bits):
    fb = (bits >> 9) | np.uint32(0x3F800000)
    f = lax.bitcast_convert_type(fb, jnp.float32) - np.float32(1.0)
    return jnp.maximum(_MINV, f * _SPAN + _MINV)


def _neg_log_unit(bits):
    return -jnp.log(_unit(bits))


def _tc_body(p_ref, o_ref, *, blk_rows, n_elems):
    g = pl.program_id(0)
    p = p_ref[...]
    row = lax.broadcasted_iota(jnp.int32, (blk_rows, _LANES), 0)
    col = lax.broadcasted_iota(jnp.int32, (blk_rows, _LANES), 1)
    base = g * (blk_rows * _LANES)
    idx = (base + row * _LANES + col).astype(jnp.uint32)
    e0 = _neg_log_unit(_threefry_bits(idx))
    e1 = _neg_log_unit(_threefry_bits(idx + np.uint32(n_elems)))
    cond = (np.float32(1.0) - p) * e0 > p * e1
    o_ref[...] = cond.astype(jnp.float32)


def _tc_part(p_full, m, n_elems):
    pad = (-p_full.shape[0]) % _LANES
    p = p_full
    if pad:
        p = jnp.pad(p, (0, pad))
    rows = p.shape[0] // _LANES
    p2 = p.reshape(rows, _LANES)
    m_rows = (m + _LANES - 1) // _LANES

    blk_rows = 512
    grid = (m_rows + blk_rows - 1) // blk_rows
    out = pl.pallas_call(
        functools.partial(_tc_body, blk_rows=blk_rows, n_elems=n_elems),
        grid=(grid,),
        in_specs=[pl.BlockSpec((blk_rows, _LANES), lambda g: (g, 0))],
        out_specs=pl.BlockSpec((blk_rows, _LANES), lambda g: (g, 0)),
        out_shape=jax.ShapeDtypeStruct((m_rows, _LANES), jnp.float32),
        compiler_params=pltpu.CompilerParams(
            dimension_semantics=("parallel",),
        ),
    )(p2)
    out = out.reshape(m_rows * _LANES)
    if m_rows * _LANES != m:
        out = out[:m]
    return out


_SC_WORKERS = 32
_SC_BLOCK = 8192
_SC_UNROLL = 4
_SC_CHUNK_BLOCKS = 1
_SC_COUNT = _SC_WORKERS * _SC_CHUNK_BLOCKS * _SC_BLOCK


def _neg_log_unit_poly(bits):
    u = _unit(bits)
    ub = lax.bitcast_convert_type(u, jnp.uint32)
    e_i = lax.bitcast_convert_type(ub >> 23, jnp.int32) - 127
    mb = (ub & np.uint32(0x007FFFFF)) | np.uint32(0x3F800000)
    z = lax.bitcast_convert_type(mb, jnp.float32) - np.float32(1.0)
    acc = jnp.full_like(z, np.float32(_LOG2_COEFFS[-1]))
    for c in _LOG2_COEFFS[-2::-1]:
        acc = acc * z + np.float32(c)
    log2u = e_i.astype(jnp.float32) + z * acc
    return -(log2u * _LN2)


def _sc_part(p_full, m, n):
    chunk = _SC_CHUNK_BLOCKS * _SC_BLOCK
    mesh = plsc.VectorSubcoreMesh(core_axis_name="c", subcore_axis_name="s")

    @functools.partial(
        pl.kernel,
        mesh=mesh,
        out_type=jax.ShapeDtypeStruct((_SC_COUNT,), jnp.float32),
        scratch_types=[
            pltpu.VMEM((_SC_BLOCK,), jnp.float32),
            pltpu.VMEM((_SC_BLOCK,), jnp.float32),
        ],
    )
    def k(p_hbm, out_hbm, p_v, o_v):
        wid = lax.axis_index("s") * 2 + lax.axis_index("c")
        woff = wid * chunk
        lane = lax.bitcast_convert_type(lax.iota(jnp.int32, 16), jnp.uint32)

        def do_block(b, carry):
            off = woff + b * _SC_BLOCK
            pltpu.sync_copy(p_hbm.at[pl.ds(m + off, _SC_BLOCK)], p_v)

            def do_vec(j, carry2):
                vbase = j * (16 * _SC_UNROLL)
                for t in range(_SC_UNROLL):
                    lo = vbase + t * 16
                    gbase = (m + off + lo).astype(jnp.uint32)
                    idx = lane + gbase
                    e0 = _neg_log_unit_poly(_threefry_bits(idx))
                    e1 = _neg_log_unit_poly(
                        _threefry_bits(idx + np.uint32(n)))
                    pv = p_v[pl.ds(lo, 16)]
                    res = jnp.where((np.float32(1.0) - pv) * e0 > pv * e1,
                                    np.float32(1.0), np.float32(0.0))
                    o_v[pl.ds(lo, 16)] = res
                return carry2

            lax.fori_loop(0, _SC_BLOCK // (16 * _SC_UNROLL), do_vec, 0)
            pltpu.sync_copy(o_v, out_hbm.at[pl.ds(off, _SC_BLOCK)])
            return carry

        lax.fori_loop(0, _SC_CHUNK_BLOCKS, do_block, 0)

    return k(p_full)


def kernel(not_infected_probs):
    n = not_infected_probs.shape[0]
    sc_count = _SC_COUNT if (n > 2 * _SC_COUNT and _SC_COUNT % _LANES == 0
                             and (n - _SC_COUNT) % _LANES == 0) else 0
    m = n - sc_count
    if not sc_count:
        return _tc_part(not_infected_probs, m, n)
    out_sc = _sc_part(not_infected_probs, m, n)
    out_tc = _tc_part(not_infected_probs, m, n)
    return jnp.concatenate([out_tc, out_sc])


# speedup vs baseline: 1.0004x; 1.0004x over previous
import functools

import numpy as np
import jax
import jax.numpy as jnp
from jax import lax
from jax.experimental import pallas as pl
from jax.experimental.pallas import tpu as pltpu
from jax.experimental.pallas import tpu_sc as plsc
from jax.experimental.compute_on import compute_on

_LANES = 128

_K0 = np.uint32(908003072)
_K1 = np.uint32(3252900185)
_K2 = np.uint32(_K0 ^ _K1 ^ np.uint32(0x1BD11BDA))
_KS = (_K0, _K1, _K2)
_ROT = ((13, 15, 26, 6), (17, 29, 16, 24))

_SPAN = np.float32(np.float32(1.0) - np.float32(1e-10))
_MINV = np.float32(1e-10)

_LOG2_COEFFS = (
    1.4426950215754873, -0.7213459745244996, 0.4808460790740989,
    -0.3599757274729275, 0.2836810758200936, -0.2202947345679977,
    0.1521123404570871, -0.0813299297601238, 0.0281939127697494,
    -0.0045820805842636,
)
_LN2 = np.float32(0.6931471805599453)


def _threefry_bits(x1v):
    x0 = jnp.full_like(x1v, _K0)
    x1 = x1v + _K1
    for i in range(5):
        for r in _ROT[i % 2]:
            x0 = x0 + x1
            x1 = (x1 << r) | (x1 >> (32 - r))
            x1 = x0 ^ x1
        x0 = x0 + _KS[(i + 1) % 3]
        x1 = x1 + np.uint32(_KS[(i + 2) % 3] + np.uint32(i + 1))
    return x0 ^ x1


def _unit(bits):
    fb = (bits >> 9) | np.uint32(0x3F800000)
    f = lax.bitcast_convert_type(fb, jnp.float32) - np.float32(1.0)
    return jnp.maximum(_MINV, f * _SPAN + _MINV)


def _neg_log_unit(bits):
    return -jnp.log(_unit(bits))


def _tc_body(p_ref, o_ref, *, blk_rows, n_elems):
    g = pl.program_id(0)
    p = p_ref[...]
    row = lax.broadcasted_iota(jnp.int32, (blk_rows, _LANES), 0)
    col = lax.broadcasted_iota(jnp.int32, (blk_rows, _LANES), 1)
    base = g * (blk_rows * _LANES)
    idx = (base + row * _LANES + col).astype(jnp.uint32)
    e0 = _neg_log_unit(_threefry_bits(idx))
    e1 = _neg_log_unit(_threefry_bits(idx + np.uint32(n_elems)))
    cond = (np.float32(1.0) - p) * e0 > p * e1
    o_ref[...] = cond.astype(jnp.float32)


def _tc_part(p_full, m, n_elems):
    pad = (-p_full.shape[0]) % _LANES
    p = p_full
    if pad:
        p = jnp.pad(p, (0, pad))
    rows = p.shape[0] // _LANES
    p2 = p.reshape(rows, _LANES)
    m_rows = (m + _LANES - 1) // _LANES

    blk_rows = 512
    grid = (m_rows + blk_rows - 1) // blk_rows
    out = pl.pallas_call(
        functools.partial(_tc_body, blk_rows=blk_rows, n_elems=n_elems),
        grid=(grid,),
        in_specs=[pl.BlockSpec((blk_rows, _LANES), lambda g: (g, 0))],
        out_specs=pl.BlockSpec((blk_rows, _LANES), lambda g: (g, 0)),
        out_shape=jax.ShapeDtypeStruct((m_rows, _LANES), jnp.float32),
        compiler_params=pltpu.CompilerParams(
            dimension_semantics=("parallel",),
        ),
    )(p2)
    out = out.reshape(m_rows * _LANES)
    if m_rows * _LANES != m:
        out = out[:m]
    return out


_SC_WORKERS = 32
_SC_BLOCK = 8192
_SC_UNROLL = 4
_SC_CHUNK_BLOCKS = 1
_SC_COUNT = _SC_WORKERS * _SC_CHUNK_BLOCKS * _SC_BLOCK


def _neg_log_unit_poly(bits):
    u = _unit(bits)
    ub = lax.bitcast_convert_type(u, jnp.uint32)
    e_i = lax.bitcast_convert_type(ub >> 23, jnp.int32) - 127
    mb = (ub & np.uint32(0x007FFFFF)) | np.uint32(0x3F800000)
    z = lax.bitcast_convert_type(mb, jnp.float32) - np.float32(1.0)
    acc = jnp.full_like(z, np.float32(_LOG2_COEFFS[-1]))
    for c in _LOG2_COEFFS[-2::-1]:
        acc = acc * z + np.float32(c)
    log2u = e_i.astype(jnp.float32) + z * acc
    return -(log2u * _LN2)


def _sc_part(p_full, m, n):
    chunk = _SC_CHUNK_BLOCKS * _SC_BLOCK
    mesh = plsc.VectorSubcoreMesh(core_axis_name="c", subcore_axis_name="s")

    @functools.partial(
        pl.kernel,
        mesh=mesh,
        out_type=jax.ShapeDtypeStruct((_SC_COUNT,), jnp.float32),
        scratch_types=[
            pltpu.VMEM((_SC_BLOCK,), jnp.float32),
            pltpu.VMEM((_SC_BLOCK,), jnp.float32),
        ],
    )
    def k(p_hbm, out_hbm, p_v, o_v):
        wid = lax.axis_index("s") * 2 + lax.axis_index("c")
        woff = wid * chunk
        lane = lax.bitcast_convert_type(lax.iota(jnp.int32, 16), jnp.uint32)

        def do_block(b, carry):
            off = woff + b * _SC_BLOCK
            pltpu.sync_copy(p_hbm.at[pl.ds(m + off, _SC_BLOCK)], p_v)

            def do_vec(j, carry2):
                vbase = j * (16 * _SC_UNROLL)
                for t in range(_SC_UNROLL):
                    lo = vbase + t * 16
                    gbase = (m + off + lo).astype(jnp.uint32)
                    idx = lane + gbase
                    e0 = _neg_log_unit_poly(_threefry_bits(idx))
                    e1 = _neg_log_unit_poly(
                        _threefry_bits(idx + np.uint32(n)))
                    pv = p_v[pl.ds(lo, 16)]
                    res = jnp.where((np.float32(1.0) - pv) * e0 > pv * e1,
                                    np.float32(1.0), np.float32(0.0))
                    o_v[pl.ds(lo, 16)] = res
                return carry2

            lax.fori_loop(0, _SC_BLOCK // (16 * _SC_UNROLL), do_vec, 0)
            pltpu.sync_copy(o_v, out_hbm.at[pl.ds(off, _SC_BLOCK)])
            return carry

        lax.fori_loop(0, _SC_CHUNK_BLOCKS, do_block, 0)

    return k(p_full)


def kernel(not_infected_probs):
    n = not_infected_probs.shape[0]
    sc_count = _SC_COUNT if (n > 2 * _SC_COUNT and _SC_COUNT % _LANES == 0
                             and (n - _SC_COUNT) % _LANES == 0) else 0
    m = n - sc_count
    if not sc_count:
        return _tc_part(not_infected_probs, m, n)
    with compute_on("tpu_sparsecore"):
        out_sc = _sc_part(not_infected_probs, m, n)
    out_tc = _tc_part(not_infected_probs, m, n)
    return jnp.concatenate([out_tc, out_sc])


# speedup vs baseline: 1.1880x; 1.1875x over previous
import functools

import numpy as np
import jax
import jax.numpy as jnp
from jax import lax
from jax.experimental import pallas as pl
from jax.experimental.pallas import tpu as pltpu
from jax.experimental.pallas import tpu_sc as plsc
from jax.experimental.compute_on import compute_on

_LANES = 128

_K0 = np.uint32(908003072)
_K1 = np.uint32(3252900185)
_K2 = np.uint32(_K0 ^ _K1 ^ np.uint32(0x1BD11BDA))
_KS = (_K0, _K1, _K2)
_ROT = ((13, 15, 26, 6), (17, 29, 16, 24))

_SPAN = np.float32(np.float32(1.0) - np.float32(1e-10))
_MINV = np.float32(1e-10)

_LOG2_COEFFS = (
    1.4426950215754873, -0.7213459745244996, 0.4808460790740989,
    -0.3599757274729275, 0.2836810758200936, -0.2202947345679977,
    0.1521123404570871, -0.0813299297601238, 0.0281939127697494,
    -0.0045820805842636,
)
_LN2 = np.float32(0.6931471805599453)


def _threefry_bits(x1v):
    x0 = jnp.full_like(x1v, _K0)
    x1 = x1v + _K1
    for i in range(5):
        for r in _ROT[i % 2]:
            x0 = x0 + x1
            x1 = (x1 << r) | (x1 >> (32 - r))
            x1 = x0 ^ x1
        x0 = x0 + _KS[(i + 1) % 3]
        x1 = x1 + np.uint32(_KS[(i + 2) % 3] + np.uint32(i + 1))
    return x0 ^ x1


def _unit(bits):
    fb = (bits >> 9) | np.uint32(0x3F800000)
    f = lax.bitcast_convert_type(fb, jnp.float32) - np.float32(1.0)
    return jnp.maximum(_MINV, f * _SPAN + _MINV)


def _neg_log_unit(bits):
    return -jnp.log(_unit(bits))


def _tc_body(p_ref, o_ref, *, blk_rows, n_elems):
    g = pl.program_id(0)
    p = p_ref[...]
    row = lax.broadcasted_iota(jnp.int32, (blk_rows, _LANES), 0)
    col = lax.broadcasted_iota(jnp.int32, (blk_rows, _LANES), 1)
    base = g * (blk_rows * _LANES)
    idx = (base + row * _LANES + col).astype(jnp.uint32)
    e0 = _neg_log_unit(_threefry_bits(idx))
    e1 = _neg_log_unit(_threefry_bits(idx + np.uint32(n_elems)))
    cond = (np.float32(1.0) - p) * e0 > p * e1
    o_ref[...] = cond.astype(jnp.float32)


def _tc_part(p_full, m, n_elems):
    pad = (-p_full.shape[0]) % _LANES
    p = p_full
    if pad:
        p = jnp.pad(p, (0, pad))
    rows = p.shape[0] // _LANES
    p2 = p.reshape(rows, _LANES)
    m_rows = (m + _LANES - 1) // _LANES

    blk_rows = 1024
    grid = (m_rows + blk_rows - 1) // blk_rows
    out = pl.pallas_call(
        functools.partial(_tc_body, blk_rows=blk_rows, n_elems=n_elems),
        grid=(grid,),
        in_specs=[pl.BlockSpec((blk_rows, _LANES), lambda g: (g, 0))],
        out_specs=pl.BlockSpec((blk_rows, _LANES), lambda g: (g, 0)),
        out_shape=jax.ShapeDtypeStruct((m_rows, _LANES), jnp.float32),
        compiler_params=pltpu.CompilerParams(
            dimension_semantics=("parallel",),
        ),
    )(p2)
    out = out.reshape(m_rows * _LANES)
    if m_rows * _LANES != m:
        out = out[:m]
    return out


_SC_WORKERS = 32
_SC_BLOCK = 8192
_SC_UNROLL = 4
_SC_CHUNK_BLOCKS = 8
_SC_COUNT = _SC_WORKERS * _SC_CHUNK_BLOCKS * _SC_BLOCK


def _neg_log_unit_poly(bits):
    u = _unit(bits)
    ub = lax.bitcast_convert_type(u, jnp.uint32)
    e_i = lax.bitcast_convert_type(ub >> 23, jnp.int32) - 127
    mb = (ub & np.uint32(0x007FFFFF)) | np.uint32(0x3F800000)
    z = lax.bitcast_convert_type(mb, jnp.float32) - np.float32(1.0)
    acc = jnp.full_like(z, np.float32(_LOG2_COEFFS[-1]))
    for c in _LOG2_COEFFS[-2::-1]:
        acc = acc * z + np.float32(c)
    log2u = e_i.astype(jnp.float32) + z * acc
    return -(log2u * _LN2)


def _sc_part(p_full, m, n):
    chunk = _SC_CHUNK_BLOCKS * _SC_BLOCK
    mesh = plsc.VectorSubcoreMesh(core_axis_name="c", subcore_axis_name="s")

    @functools.partial(
        pl.kernel,
        mesh=mesh,
        out_type=jax.ShapeDtypeStruct((_SC_COUNT,), jnp.float32),
        scratch_types=[
            pltpu.VMEM((_SC_BLOCK,), jnp.float32),
            pltpu.VMEM((_SC_BLOCK,), jnp.float32),
        ],
    )
    def k(p_hbm, out_hbm, p_v, o_v):
        wid = lax.axis_index("s") * 2 + lax.axis_index("c")
        woff = wid * chunk
        lane = lax.bitcast_convert_type(lax.iota(jnp.int32, 16), jnp.uint32)

        def do_block(b, carry):
            off = woff + b * _SC_BLOCK
            pltpu.sync_copy(p_hbm.at[pl.ds(m + off, _SC_BLOCK)], p_v)

            def do_vec(j, carry2):
                vbase = j * (16 * _SC_UNROLL)
                for t in range(_SC_UNROLL):
                    lo = vbase + t * 16
                    gbase = (m + off + lo).astype(jnp.uint32)
                    idx = lane + gbase
                    e0 = _neg_log_unit_poly(_threefry_bits(idx))
                    e1 = _neg_log_unit_poly(
                        _threefry_bits(idx + np.uint32(n)))
                    pv = p_v[pl.ds(lo, 16)]
                    res = jnp.where((np.float32(1.0) - pv) * e0 > pv * e1,
                                    np.float32(1.0), np.float32(0.0))
                    o_v[pl.ds(lo, 16)] = res
                return carry2

            lax.fori_loop(0, _SC_BLOCK // (16 * _SC_UNROLL), do_vec, 0)
            pltpu.sync_copy(o_v, out_hbm.at[pl.ds(off, _SC_BLOCK)])
            return carry

        lax.fori_loop(0, _SC_CHUNK_BLOCKS, do_block, 0)

    return k(p_full)


def kernel(not_infected_probs):
    n = not_infected_probs.shape[0]
    sc_count = _SC_COUNT if (n > 2 * _SC_COUNT and _SC_COUNT % _LANES == 0
                             and (n - _SC_COUNT) % _LANES == 0) else 0
    m = n - sc_count
    if not sc_count:
        return _tc_part(not_infected_probs, m, n)
    with compute_on("tpu_sparsecore"):
        out_sc = _sc_part(not_infected_probs, m, n)
    out_tc = _tc_part(not_infected_probs, m, n)
    return jnp.concatenate([out_tc, out_sc])


# speedup vs baseline: 1.1941x; 1.0051x over previous
import functools

import numpy as np
import jax
import jax.numpy as jnp
from jax import lax
from jax.experimental import pallas as pl
from jax.experimental.pallas import tpu as pltpu
from jax.experimental.pallas import tpu_sc as plsc
from jax.experimental.compute_on import compute_on

_LANES = 128

_K0 = np.uint32(908003072)
_K1 = np.uint32(3252900185)
_K2 = np.uint32(_K0 ^ _K1 ^ np.uint32(0x1BD11BDA))
_KS = (_K0, _K1, _K2)
_ROT = ((13, 15, 26, 6), (17, 29, 16, 24))

_SPAN = np.float32(np.float32(1.0) - np.float32(1e-10))
_MINV = np.float32(1e-10)

_LOG2_COEFFS = (
    1.4426950215754873, -0.7213459745244996, 0.4808460790740989,
    -0.3599757274729275, 0.2836810758200936, -0.2202947345679977,
    0.1521123404570871, -0.0813299297601238, 0.0281939127697494,
    -0.0045820805842636,
)
_LN2 = np.float32(0.6931471805599453)


def _threefry_bits(x1v):
    x0 = jnp.full_like(x1v, _K0)
    x1 = x1v + _K1
    for i in range(5):
        for r in _ROT[i % 2]:
            x0 = x0 + x1
            x1 = (x1 << r) | (x1 >> (32 - r))
            x1 = x0 ^ x1
        x0 = x0 + _KS[(i + 1) % 3]
        x1 = x1 + np.uint32(_KS[(i + 2) % 3] + np.uint32(i + 1))
    return x0 ^ x1


def _unit(bits):
    fb = (bits >> 9) | np.uint32(0x3F800000)
    f = lax.bitcast_convert_type(fb, jnp.float32) - np.float32(1.0)
    return jnp.maximum(_MINV, f * _SPAN + _MINV)


def _neg_log_unit(bits):
    return -jnp.log(_unit(bits))


def _tc_body(p_ref, o_ref, *, blk_rows, n_elems):
    g = pl.program_id(0)
    p = p_ref[...]
    row = lax.broadcasted_iota(jnp.int32, (blk_rows, _LANES), 0)
    col = lax.broadcasted_iota(jnp.int32, (blk_rows, _LANES), 1)
    base = g * (blk_rows * _LANES)
    idx = (base + row * _LANES + col).astype(jnp.uint32)
    e0 = _neg_log_unit(_threefry_bits(idx))
    e1 = _neg_log_unit(_threefry_bits(idx + np.uint32(n_elems)))
    cond = (np.float32(1.0) - p) * e0 > p * e1
    o_ref[...] = cond.astype(jnp.float32)


def _tc_part(p_full, m, n_elems):
    pad = (-p_full.shape[0]) % _LANES
    p = p_full
    if pad:
        p = jnp.pad(p, (0, pad))
    rows = p.shape[0] // _LANES
    p2 = p.reshape(rows, _LANES)
    m_rows = (m + _LANES - 1) // _LANES

    blk_rows = 2048
    grid = (m_rows + blk_rows - 1) // blk_rows
    out = pl.pallas_call(
        functools.partial(_tc_body, blk_rows=blk_rows, n_elems=n_elems),
        grid=(grid,),
        in_specs=[pl.BlockSpec((blk_rows, _LANES), lambda g: (g, 0))],
        out_specs=pl.BlockSpec((blk_rows, _LANES), lambda g: (g, 0)),
        out_shape=jax.ShapeDtypeStruct((m_rows, _LANES), jnp.float32),
        compiler_params=pltpu.CompilerParams(
            dimension_semantics=("parallel",),
        ),
    )(p2)
    out = out.reshape(m_rows * _LANES)
    if m_rows * _LANES != m:
        out = out[:m]
    return out


_SC_WORKERS = 32
_SC_BLOCK = 8192
_SC_UNROLL = 8
_SC_CHUNK_BLOCKS = 8
_SC_COUNT = _SC_WORKERS * _SC_CHUNK_BLOCKS * _SC_BLOCK


def _neg_log_unit_poly(bits):
    u = _unit(bits)
    ub = lax.bitcast_convert_type(u, jnp.uint32)
    e_i = lax.bitcast_convert_type(ub >> 23, jnp.int32) - 127
    mb = (ub & np.uint32(0x007FFFFF)) | np.uint32(0x3F800000)
    z = lax.bitcast_convert_type(mb, jnp.float32) - np.float32(1.0)
    acc = jnp.full_like(z, np.float32(_LOG2_COEFFS[-1]))
    for c in _LOG2_COEFFS[-2::-1]:
        acc = acc * z + np.float32(c)
    log2u = e_i.astype(jnp.float32) + z * acc
    return -(log2u * _LN2)


def _sc_part(p_full, m, n):
    chunk = _SC_CHUNK_BLOCKS * _SC_BLOCK
    mesh = plsc.VectorSubcoreMesh(core_axis_name="c", subcore_axis_name="s")

    @functools.partial(
        pl.kernel,
        mesh=mesh,
        out_type=jax.ShapeDtypeStruct((_SC_COUNT,), jnp.float32),
        scratch_types=[
            pltpu.VMEM((_SC_BLOCK,), jnp.float32),
            pltpu.VMEM((_SC_BLOCK,), jnp.float32),
        ],
    )
    def k(p_hbm, out_hbm, p_v, o_v):
        wid = lax.axis_index("s") * 2 + lax.axis_index("c")
        woff = wid * chunk
        lane = lax.bitcast_convert_type(lax.iota(jnp.int32, 16), jnp.uint32)

        def do_block(b, carry):
            off = woff + b * _SC_BLOCK
            pltpu.sync_copy(p_hbm.at[pl.ds(m + off, _SC_BLOCK)], p_v)

            def do_vec(j, carry2):
                vbase = j * (16 * _SC_UNROLL)
                for t in range(_SC_UNROLL):
                    lo = vbase + t * 16
                    gbase = (m + off + lo).astype(jnp.uint32)
                    idx = lane + gbase
                    e0 = _neg_log_unit_poly(_threefry_bits(idx))
                    e1 = _neg_log_unit_poly(
                        _threefry_bits(idx + np.uint32(n)))
                    pv = p_v[pl.ds(lo, 16)]
                    res = jnp.where((np.float32(1.0) - pv) * e0 > pv * e1,
                                    np.float32(1.0), np.float32(0.0))
                    o_v[pl.ds(lo, 16)] = res
                return carry2

            lax.fori_loop(0, _SC_BLOCK // (16 * _SC_UNROLL), do_vec, 0)
            pltpu.sync_copy(o_v, out_hbm.at[pl.ds(off, _SC_BLOCK)])
            return carry

        lax.fori_loop(0, _SC_CHUNK_BLOCKS, do_block, 0)

    return k(p_full)


def kernel(not_infected_probs):
    n = not_infected_probs.shape[0]
    sc_count = _SC_COUNT if (n > 2 * _SC_COUNT and _SC_COUNT % _LANES == 0
                             and (n - _SC_COUNT) % _LANES == 0) else 0
    m = n - sc_count
    if not sc_count:
        return _tc_part(not_infected_probs, m, n)
    with compute_on("tpu_sparsecore"):
        out_sc = _sc_part(not_infected_probs, m, n)
    out_tc = _tc_part(not_infected_probs, m, n)
    return jnp.concatenate([out_tc, out_sc])
